# Initial kernel scaffold; baseline (speedup 1.0000x reference)
import jax
import jax.numpy as jnp
from jax.experimental import pallas as pl


def kernel(point_cloud, W_rel1, b1, W_root1, W_rel2, b2, W_root2, W_rel3, b3, W_root3):
    raise NotImplementedError("write your pallas kernel here")


# speedup vs baseline: 5.7301x; 5.7301x over previous
import functools

import jax
import jax.numpy as jnp
from jax.experimental import pallas as pl
from jax.experimental.pallas import tpu as pltpu

B = 8
N = 6250
K = 16
NP = 6272
R = 128
NB = NP // R
CH = 896
NCH = NP // CH
DP = 8

_BIG = 2 ** 30


def _d2_chunk(a, sq_row, ptsT_ref, c):
    btc = ptsT_ref[0, :, c * CH:(c + 1) * CH]
    sq_c = jnp.sum(btc * btc, axis=0)
    prod = jax.lax.dot_general(
        a, btc, (((1,), (0,)), ((), ())),
        preferred_element_type=jnp.float32)
    d2c = sq_row[:, None] + sq_c[None, :] - 2.0 * prod
    colc = jax.lax.broadcasted_iota(jnp.int32, (R, CH), 1) + c * CH
    return d2c, colc


def _knn_kernel(pts_ref, ptsT_ref, idx_ref, t_ref, it_ref, d2_ref):
    a = pts_ref[0]
    sq_row = jnp.sum(a * a, axis=1)
    inf = jnp.float32(jnp.inf)
    for c in range(NCH):
        d2c, colc = _d2_chunk(a, sq_row, ptsT_ref, c)
        d2_ref[:, c * CH:(c + 1) * CH] = jnp.where(colc < N, d2c, inf)

    idx_cols = []
    v = None
    ii = None
    for _ in range(K):
        v = jnp.full((R,), jnp.inf, jnp.float32)
        for c in range(NCH):
            v = jnp.minimum(v, jnp.min(d2_ref[:, c * CH:(c + 1) * CH], axis=1))
        ii = jnp.full((R,), _BIG, jnp.int32)
        for c in range(NCH):
            d2c = d2_ref[:, c * CH:(c + 1) * CH]
            colc = jax.lax.broadcasted_iota(jnp.int32, (R, CH), 1) + c * CH
            cand = jnp.where(d2c == v[:, None], colc, _BIG)
            ii = jnp.minimum(ii, jnp.min(cand, axis=1))
        for c in range(NCH):
            d2c = d2_ref[:, c * CH:(c + 1) * CH]
            colc = jax.lax.broadcasted_iota(jnp.int32, (R, CH), 1) + c * CH
            hit = (d2c == v[:, None]) & (colc == ii[:, None])
            d2_ref[:, c * CH:(c + 1) * CH] = jnp.where(hit, inf, d2c)
        idx_cols.append(ii)

    idx_ref[0] = jnp.stack(idx_cols, axis=1)
    t_ref[0, 0, :] = v
    it_ref[0, 0, :] = ii


def _layer_kernel(pts_ref, ptsT_ref, t_ref, it_ref, xf_ref, wrel_ref, b_ref,
                  wroot_ref, out_ref, *, di, do, relu):
    a = pts_ref[0]
    sq_row = jnp.sum(a * a, axis=1)
    t = t_ref[0, 0, :]
    it = it_ref[0, 0, :]
    aggr = jnp.zeros((R, di), jnp.float32)
    for c in range(NCH):
        d2c, colc = _d2_chunk(a, sq_row, ptsT_ref, c)
        m = (d2c < t[:, None]) | ((d2c == t[:, None]) & (colc <= it[:, None]))
        m = m & (colc < N)
        mf = jnp.where(m, jnp.float32(1.0), jnp.float32(0.0))
        xc = xf_ref[0, c * CH:(c + 1) * CH, :]
        aggr = aggr + jax.lax.dot_general(
            mf, xc, (((1,), (0,)), ((), ())),
            preferred_element_type=jnp.float32)
    i = pl.program_id(1)
    xrows = xf_ref[0, pl.ds(i * R, R), :]
    out = (jax.lax.dot_general(aggr, wrel_ref[...],
                               (((1,), (0,)), ((), ())),
                               preferred_element_type=jnp.float32)
           + b_ref[0, :][None, :]
           + jax.lax.dot_general(xrows, wroot_ref[...],
                                 (((1,), (0,)), ((), ())),
                                 preferred_element_type=jnp.float32))
    if relu:
        out = jnp.maximum(out, 0.0)
    out_ref[0] = out


def _run_knn(pts_blk, ptsT):
    grid = (B, NB)
    return pl.pallas_call(
        _knn_kernel,
        grid=grid,
        in_specs=[
            pl.BlockSpec((1, R, DP), lambda b, i: (b * NB + i, 0, 0)),
            pl.BlockSpec((1, DP, NP), lambda b, i: (b, 0, 0)),
        ],
        out_specs=[
            pl.BlockSpec((1, R, K), lambda b, i: (b * NB + i, 0, 0)),
            pl.BlockSpec((1, 1, R), lambda b, i: (b * NB + i, 0, 0)),
            pl.BlockSpec((1, 1, R), lambda b, i: (b * NB + i, 0, 0)),
        ],
        out_shape=[
            jax.ShapeDtypeStruct((B * NB, R, K), jnp.int32),
            jax.ShapeDtypeStruct((B * NB, 1, R), jnp.float32),
            jax.ShapeDtypeStruct((B * NB, 1, R), jnp.int32),
        ],
        scratch_shapes=[pltpu.VMEM((R, NP), jnp.float32)],
    )(pts_blk, ptsT)


def _run_layer(pts_blk, ptsT, t, it, xf, wrel, bvec, wroot, *, relu):
    di = xf.shape[-1]
    do = wrel.shape[-1]
    grid = (B, NB)
    body = functools.partial(_layer_kernel, di=di, do=do, relu=relu)
    return pl.pallas_call(
        body,
        grid=grid,
        in_specs=[
            pl.BlockSpec((1, R, DP), lambda b, i: (b * NB + i, 0, 0)),
            pl.BlockSpec((1, DP, NP), lambda b, i: (b, 0, 0)),
            pl.BlockSpec((1, 1, R), lambda b, i: (b * NB + i, 0, 0)),
            pl.BlockSpec((1, 1, R), lambda b, i: (b * NB + i, 0, 0)),
            pl.BlockSpec((1, NP, di), lambda b, i: (b, 0, 0)),
            pl.BlockSpec((di, do), lambda b, i: (0, 0)),
            pl.BlockSpec((1, do), lambda b, i: (0, 0)),
            pl.BlockSpec((di, do), lambda b, i: (0, 0)),
        ],
        out_specs=pl.BlockSpec((1, R, do), lambda b, i: (b, i, 0)),
        out_shape=jax.ShapeDtypeStruct((B, NP, do), jnp.float32),
    )(pts_blk, ptsT, t, it, xf, wrel, bvec, wroot)


def kernel(point_cloud, W_rel1, b1, W_root1, W_rel2, b2, W_root2,
           W_rel3, b3, W_root3):
    pts_pad = jnp.pad(point_cloud, ((0, 0), (0, NP - N), (0, DP - 3)))
    ptsT = jnp.transpose(pts_pad, (0, 2, 1))
    pts_blk = pts_pad.reshape(B * NB, R, DP)

    _, t, it = _run_knn(pts_blk, ptsT)

    w1 = jnp.pad(W_rel1, ((0, DP - 3), (0, 0)))
    wr1 = jnp.pad(W_root1, ((0, DP - 3), (0, 0)))

    x1 = _run_layer(pts_blk, ptsT, t, it, pts_pad, w1, b1[None, :], wr1,
                    relu=True)
    x2 = _run_layer(pts_blk, ptsT, t, it, x1, W_rel2, b2[None, :], W_root2,
                    relu=True)
    x3 = _run_layer(pts_blk, ptsT, t, it, x2, W_rel3, b3[None, :], W_root3,
                    relu=False)
    return x3[:, :N, :]


# speedup vs baseline: 6.7131x; 1.1716x over previous
import functools

import jax
import jax.numpy as jnp
from jax import lax
from jax.experimental import pallas as pl
from jax.experimental.pallas import tpu as pltpu
from jax.experimental.pallas import tpu_sc as plsc

B = 8
N = 6250
K = 16
NP = 6272
NT = B * NP
R = 128
NB = NP // R
CH = 896
NCH = NP // CH
DP = 8

NW = 32
PER_W = NT // NW
G = 8

_BIG = 2 ** 30


def _d2_chunk(a, sq_row, ptsT_ref, c):
    btc = ptsT_ref[0, :, c * CH:(c + 1) * CH]
    sq_c = jnp.sum(btc * btc, axis=0)
    prod = jax.lax.dot_general(
        a, btc, (((1,), (0,)), ((), ())),
        preferred_element_type=jnp.float32)
    d2c = sq_row[:, None] + sq_c[None, :] - 2.0 * prod
    colc = jax.lax.broadcasted_iota(jnp.int32, (R, CH), 1) + c * CH
    return d2c, colc


def _knn_kernel(pts_ref, ptsT_ref, idx_ref, d2_ref):
    a = pts_ref[0]
    sq_row = jnp.sum(a * a, axis=1)
    inf = jnp.float32(jnp.inf)
    for c in range(NCH):
        d2c, colc = _d2_chunk(a, sq_row, ptsT_ref, c)
        d2_ref[:, c * CH:(c + 1) * CH] = jnp.where(colc < N, d2c, inf)

    iota = jax.lax.broadcasted_iota(jnp.int32, (R, CH), 1)
    idx_cols = []
    for _ in range(K):
        m = d2_ref[:, 0:CH]
        for c in range(1, NCH):
            m = jnp.minimum(m, d2_ref[:, c * CH:(c + 1) * CH])
        v = jnp.min(m, axis=1)
        cand = jnp.full((R, CH), _BIG, jnp.int32)
        for c in range(NCH):
            d2c = d2_ref[:, c * CH:(c + 1) * CH]
            cand = jnp.minimum(
                cand, jnp.where(d2c == v[:, None], iota + c * CH, _BIG))
        ii = jnp.min(cand, axis=1)
        for c in range(NCH):
            d2c = d2_ref[:, c * CH:(c + 1) * CH]
            hit = (iota + c * CH) == ii[:, None]
            d2_ref[:, c * CH:(c + 1) * CH] = jnp.where(hit, inf, d2c)
        idx_cols.append(ii)

    b = pl.program_id(0)
    idx_ref[0] = jnp.stack(idx_cols, axis=1) + b * NP


def _run_knn(pts_blk, ptsT):
    return pl.pallas_call(
        _knn_kernel,
        grid=(B, NB),
        in_specs=[
            pl.BlockSpec((1, R, DP), lambda b, i: (b * NB + i, 0, 0)),
            pl.BlockSpec((1, DP, NP), lambda b, i: (b, 0, 0)),
        ],
        out_specs=pl.BlockSpec((1, R, K), lambda b, i: (b * NB + i, 0, 0)),
        out_shape=jax.ShapeDtypeStruct((B * NB, R, K), jnp.int32),
        scratch_shapes=[pltpu.VMEM((R, NP), jnp.float32)],
    )(pts_blk, ptsT)


def _dense_kernel(x_ref, wrel_ref, b_ref, wroot_ref, z_ref, r_ref):
    x = x_ref[...]
    z_ref[...] = jax.lax.dot_general(
        x, wrel_ref[...], (((1,), (0,)), ((), ())),
        preferred_element_type=jnp.float32)
    r_ref[...] = (jax.lax.dot_general(
        x, wroot_ref[...], (((1,), (0,)), ((), ())),
        preferred_element_type=jnp.float32) + b_ref[0, :][None, :])


RZ = 512


def _run_dense(x, wrel, bvec, wroot):
    di = x.shape[-1]
    do = wrel.shape[-1]
    return pl.pallas_call(
        _dense_kernel,
        grid=(NT // RZ,),
        in_specs=[
            pl.BlockSpec((RZ, di), lambda i: (i, 0)),
            pl.BlockSpec((di, do), lambda i: (0, 0)),
            pl.BlockSpec((1, do), lambda i: (0, 0)),
            pl.BlockSpec((di, do), lambda i: (0, 0)),
        ],
        out_specs=[
            pl.BlockSpec((RZ, do), lambda i: (i, 0)),
            pl.BlockSpec((RZ, do), lambda i: (i, 0)),
        ],
        out_shape=[
            jax.ShapeDtypeStruct((NT, do), jnp.float32),
            jax.ShapeDtypeStruct((NT, do), jnp.float32),
        ],
    )(x, wrel, bvec, wroot)


DO = 128


def _gather_kernel(idx_hbm, z_hbm, r_hbm, out_hbm, idxv, rows, rbuf, obuf,
                   sem, *, do, relu):
    wid = lax.axis_index("s") * 2 + lax.axis_index("c")
    nd = do // 16

    for g in range(G):
        for d in range(nd, DO // 16):
            obuf[g, d * 16:(d + 1) * 16] = jnp.zeros((16,), jnp.float32)

    def step(it, carry):
        base = wid * PER_W + it * G
        pltpu.sync_copy(idx_hbm.at[pl.ds(base * K, G * K)], idxv)
        pltpu.async_copy(z_hbm.at[idxv], rows, sem).wait()
        pltpu.sync_copy(r_hbm.at[pl.ds(base, G)], rbuf)
        for g in range(G):
            for d in range(nd):
                acc = rbuf[g, d * 16:(d + 1) * 16]
                for j in range(K):
                    acc = acc + rows[g * K + j, d * 16:(d + 1) * 16]
                if relu:
                    acc = jnp.maximum(acc, 0.0)
                obuf[g, d * 16:(d + 1) * 16] = acc
        pltpu.sync_copy(obuf, out_hbm.at[pl.ds(base, G)])
        return carry

    lax.fori_loop(0, PER_W // G, step, 0)


def _run_gather(idx_flat, z, r, *, do, relu):
    mesh = plsc.VectorSubcoreMesh(core_axis_name="c", subcore_axis_name="s")
    body = functools.partial(_gather_kernel, do=do, relu=relu)
    f = functools.partial(
        pl.kernel, mesh=mesh,
        out_type=jax.ShapeDtypeStruct((NT, DO), jnp.float32),
        scratch_types=[
            pltpu.VMEM((G * K,), jnp.int32),
            pltpu.VMEM((G * K, DO), jnp.float32),
            pltpu.VMEM((G, DO), jnp.float32),
            pltpu.VMEM((G, DO), jnp.float32),
            pltpu.SemaphoreType.DMA,
        ],
    )(body)
    return f(idx_flat, z, r)


def kernel(point_cloud, W_rel1, b1, W_root1, W_rel2, b2, W_root2,
           W_rel3, b3, W_root3):
    pts_pad = jnp.pad(point_cloud, ((0, 0), (0, NP - N), (0, DP - 3)))
    ptsT = jnp.transpose(pts_pad, (0, 2, 1))
    pts_blk = pts_pad.reshape(B * NB, R, DP)

    idx = _run_knn(pts_blk, ptsT)
    idx_flat = idx.reshape(NT * K)

    def padw(w, dip):
        return jnp.pad(w, ((0, dip - w.shape[0]), (0, DO - w.shape[1])))

    layers = (
        (padw(W_rel1, DP), jnp.pad(b1, (0, DO - b1.shape[0])),
         padw(W_root1, DP), True),
        (padw(W_rel2, DO), jnp.pad(b2, (0, DO - b2.shape[0])),
         padw(W_root2, DO), True),
        (padw(W_rel3, DO), jnp.pad(b3, (0, DO - b3.shape[0])),
         padw(W_root3, DO), False),
    )
    dos = (64, 64, 128)

    x = pts_pad.reshape(NT, DP)
    for (wrel, bvec, wroot, relu), do in zip(layers, dos):
        z, r = _run_dense(x, wrel, bvec[None, :], wroot)
        x = _run_gather(idx_flat, z, r, do=do, relu=relu)
    return x.reshape(B, NP, DO)[:, :N, :]


# speedup vs baseline: 7.0690x; 1.0530x over previous
import functools

import jax
import jax.numpy as jnp
from jax import lax
from jax.experimental import pallas as pl
from jax.experimental.pallas import tpu as pltpu
from jax.experimental.pallas import tpu_sc as plsc

B = 8
N = 6250
K = 16
NP = 6272
NT = B * NP
R = 128
NB = NP // R
CH = 896
NCH = NP // CH
DP = 8

NW = 32
PER_W = NT // NW
G = 8

_BIG = 2 ** 30


def _d2_chunk(a, sq_row, ptsT_ref, c):
    btc = ptsT_ref[0, :, c * CH:(c + 1) * CH]
    sq_c = jnp.sum(btc * btc, axis=0)
    prod = jax.lax.dot_general(
        a, btc, (((1,), (0,)), ((), ())),
        preferred_element_type=jnp.float32)
    d2c = sq_row[:, None] + sq_c[None, :] - 2.0 * prod
    colc = jax.lax.broadcasted_iota(jnp.int32, (R, CH), 1) + c * CH
    return d2c, colc


def _knn_kernel(pts_ref, ptsT_ref, idx_ref, d2_ref):
    a = pts_ref[0]
    sq_row = jnp.sum(a * a, axis=1)
    inf = jnp.float32(jnp.inf)
    for c in range(NCH):
        d2c, colc = _d2_chunk(a, sq_row, ptsT_ref, c)
        d2_ref[:, c * CH:(c + 1) * CH] = jnp.where(colc < N, d2c, inf)

    iota = jax.lax.broadcasted_iota(jnp.int32, (R, CH), 1)
    idx_cols = []
    for _ in range(K):
        m = d2_ref[:, 0:CH]
        for c in range(1, NCH):
            m = jnp.minimum(m, d2_ref[:, c * CH:(c + 1) * CH])
        v = jnp.min(m, axis=1)
        cand = jnp.full((R, CH), _BIG, jnp.int32)
        for c in range(NCH):
            d2c = d2_ref[:, c * CH:(c + 1) * CH]
            cand = jnp.minimum(
                cand, jnp.where(d2c == v[:, None], iota + c * CH, _BIG))
        ii = jnp.min(cand, axis=1)
        for c in range(NCH):
            d2c = d2_ref[:, c * CH:(c + 1) * CH]
            hit = (iota + c * CH) == ii[:, None]
            d2_ref[:, c * CH:(c + 1) * CH] = jnp.where(hit, inf, d2c)
        idx_cols.append(ii)

    b = pl.program_id(0)
    idx_ref[0] = jnp.stack(idx_cols, axis=1) + b * NP


def _run_knn(pts_blk, ptsT):
    return pl.pallas_call(
        _knn_kernel,
        grid=(B, NB),
        in_specs=[
            pl.BlockSpec((1, R, DP), lambda b, i: (b * NB + i, 0, 0)),
            pl.BlockSpec((1, DP, NP), lambda b, i: (b, 0, 0)),
        ],
        out_specs=pl.BlockSpec((1, R, K), lambda b, i: (b * NB + i, 0, 0)),
        out_shape=jax.ShapeDtypeStruct((B * NB, R, K), jnp.int32),
        scratch_shapes=[pltpu.VMEM((R, NP), jnp.float32)],
    )(pts_blk, ptsT)


def _dense_kernel(x_ref, wrel_ref, b_ref, wroot_ref, z_ref, r_ref):
    x = x_ref[...]
    z_ref[...] = jax.lax.dot_general(
        x, wrel_ref[...], (((1,), (0,)), ((), ())),
        preferred_element_type=jnp.float32)
    r_ref[...] = (jax.lax.dot_general(
        x, wroot_ref[...], (((1,), (0,)), ((), ())),
        preferred_element_type=jnp.float32) + b_ref[0, :][None, :])


RZ = 512


def _run_dense(x, wrel, bvec, wroot):
    di = x.shape[-1]
    do = wrel.shape[-1]
    return pl.pallas_call(
        _dense_kernel,
        grid=(NT // RZ,),
        in_specs=[
            pl.BlockSpec((RZ, di), lambda i: (i, 0)),
            pl.BlockSpec((di, do), lambda i: (0, 0)),
            pl.BlockSpec((1, do), lambda i: (0, 0)),
            pl.BlockSpec((di, do), lambda i: (0, 0)),
        ],
        out_specs=[
            pl.BlockSpec((RZ, do), lambda i: (i, 0)),
            pl.BlockSpec((RZ, do), lambda i: (i, 0)),
        ],
        out_shape=[
            jax.ShapeDtypeStruct((NT, do), jnp.float32),
            jax.ShapeDtypeStruct((NT, do), jnp.float32),
        ],
    )(x, wrel, bvec, wroot)


DO = 128


def _gather_kernel(idx_hbm, z_hbm, r_hbm, out_hbm, idxv0, idxv1, rows0, rows1,
                   rbuf, obuf, sem0, sem1, *, do, relu):
    wid = lax.axis_index("s") * 2 + lax.axis_index("c")
    nd = do // 16
    nstep = PER_W // G
    npair = nstep // 2

    for g in range(G):
        for d in range(nd, DO // 16):
            obuf[g, d * 16:(d + 1) * 16] = jnp.zeros((16,), jnp.float32)

    def start(s, idxv, rows, sem):
        base = wid * PER_W + s * G
        pltpu.sync_copy(idx_hbm.at[pl.ds(base * K, G * K)], idxv)
        pltpu.async_copy(z_hbm.at[idxv], rows, sem)

    def finish(s, idxv, rows, sem):
        base = wid * PER_W + s * G
        pltpu.make_async_copy(z_hbm.at[idxv], rows, sem).wait()
        pltpu.sync_copy(r_hbm.at[pl.ds(base, G)], rbuf)
        for g in range(G):
            for d in range(nd):
                acc = rbuf[g, d * 16:(d + 1) * 16]
                for j in range(K):
                    acc = acc + rows[g * K + j, d * 16:(d + 1) * 16]
                if relu:
                    acc = jnp.maximum(acc, 0.0)
                obuf[g, d * 16:(d + 1) * 16] = acc
        pltpu.sync_copy(obuf, out_hbm.at[pl.ds(base, G)])

    start(0, idxv0, rows0, sem0)

    def pair(it, carry):
        s0 = it * 2
        start(s0 + 1, idxv1, rows1, sem1)
        finish(s0, idxv0, rows0, sem0)

        @pl.when(it + 1 < npair)
        def _():
            start(s0 + 2, idxv0, rows0, sem0)

        finish(s0 + 1, idxv1, rows1, sem1)
        return carry

    lax.fori_loop(0, npair, pair, 0)


def _run_gather(idx_flat, z, r, *, do, relu):
    mesh = plsc.VectorSubcoreMesh(core_axis_name="c", subcore_axis_name="s")
    body = functools.partial(_gather_kernel, do=do, relu=relu)
    f = functools.partial(
        pl.kernel, mesh=mesh,
        out_type=jax.ShapeDtypeStruct((NT, DO), jnp.float32),
        scratch_types=[
            pltpu.VMEM((G * K,), jnp.int32),
            pltpu.VMEM((G * K,), jnp.int32),
            pltpu.VMEM((G * K, DO), jnp.float32),
            pltpu.VMEM((G * K, DO), jnp.float32),
            pltpu.VMEM((G, DO), jnp.float32),
            pltpu.VMEM((G, DO), jnp.float32),
            pltpu.SemaphoreType.DMA,
            pltpu.SemaphoreType.DMA,
        ],
    )(body)
    return f(idx_flat, z, r)


def kernel(point_cloud, W_rel1, b1, W_root1, W_rel2, b2, W_root2,
           W_rel3, b3, W_root3):
    pts_pad = jnp.pad(point_cloud, ((0, 0), (0, NP - N), (0, DP - 3)))
    ptsT = jnp.transpose(pts_pad, (0, 2, 1))
    pts_blk = pts_pad.reshape(B * NB, R, DP)

    idx = _run_knn(pts_blk, ptsT)
    idx_flat = idx.reshape(NT * K)

    def padw(w, dip):
        return jnp.pad(w, ((0, dip - w.shape[0]), (0, DO - w.shape[1])))

    layers = (
        (padw(W_rel1, DP), jnp.pad(b1, (0, DO - b1.shape[0])),
         padw(W_root1, DP), True),
        (padw(W_rel2, DO), jnp.pad(b2, (0, DO - b2.shape[0])),
         padw(W_root2, DO), True),
        (padw(W_rel3, DO), jnp.pad(b3, (0, DO - b3.shape[0])),
         padw(W_root3, DO), False),
    )
    dos = (64, 64, 128)

    x = pts_pad.reshape(NT, DP)
    for (wrel, bvec, wroot, relu), do in zip(layers, dos):
        z, r = _run_dense(x, wrel, bvec[None, :], wroot)
        x = _run_gather(idx_flat, z, r, do=do, relu=relu)
    return x.reshape(B, NP, DO)[:, :N, :]


# speedup vs baseline: 9.5131x; 1.3457x over previous
import functools

import jax
import jax.numpy as jnp
from jax import lax
from jax.experimental import pallas as pl
from jax.experimental.pallas import tpu as pltpu
from jax.experimental.pallas import tpu_sc as plsc

B = 8
N = 6250
K = 16
NP = 6272
NT = B * NP
R = 128
NB = NP // R
CH = 896
NCH = NP // CH
DP = 8

NW = 32
PER_W = NT // NW
G = 8

_BIG = 2 ** 30


def _d2_chunk(a, sq_row, ptsT_ref, c):
    btc = ptsT_ref[0, :, c * CH:(c + 1) * CH]
    sq_c = jnp.sum(btc * btc, axis=0)
    prod = jax.lax.dot_general(
        a, btc, (((1,), (0,)), ((), ())),
        preferred_element_type=jnp.float32)
    d2c = sq_row[:, None] + sq_c[None, :] - 2.0 * prod
    colc = jax.lax.broadcasted_iota(jnp.int32, (R, CH), 1) + c * CH
    return d2c, colc


def _knn_kernel(pts_ref, ptsT_ref, idx_ref, d2_ref):
    a = pts_ref[0]
    sq_row = jnp.sum(a * a, axis=1)
    inf = jnp.float32(jnp.inf)
    for c in range(NCH):
        d2c, colc = _d2_chunk(a, sq_row, ptsT_ref, c)
        d2_ref[:, c * CH:(c + 1) * CH] = jnp.where(colc < N, d2c, inf)

    iotaf = jax.lax.broadcasted_iota(jnp.int32, (R, CH), 1).astype(jnp.float32)
    bigf = jnp.float32(float(_BIG))
    idx_cols = []
    for _ in range(K):
        m = d2_ref[:, 0:CH]
        for c in range(1, NCH):
            m = jnp.minimum(m, d2_ref[:, c * CH:(c + 1) * CH])
        v = jnp.min(m, axis=1)
        cand = jnp.full((R, CH), bigf, jnp.float32)
        for c in range(NCH):
            d2c = d2_ref[:, c * CH:(c + 1) * CH]
            cand = jnp.minimum(
                cand, jnp.where(d2c == v[:, None], iotaf + float(c * CH),
                                bigf))
        iif = jnp.min(cand, axis=1)
        for c in range(NCH):
            d2c = d2_ref[:, c * CH:(c + 1) * CH]
            hit = (iotaf + float(c * CH)) == iif[:, None]
            d2_ref[:, c * CH:(c + 1) * CH] = jnp.where(hit, inf, d2c)
        idx_cols.append(iif.astype(jnp.int32))

    b = pl.program_id(0)
    idx_ref[0] = jnp.stack(idx_cols, axis=1) + b * NP


def _run_knn(pts_blk, ptsT):
    return pl.pallas_call(
        _knn_kernel,
        grid=(B, NB),
        in_specs=[
            pl.BlockSpec((1, R, DP), lambda b, i: (b * NB + i, 0, 0)),
            pl.BlockSpec((1, DP, NP), lambda b, i: (b, 0, 0)),
        ],
        out_specs=pl.BlockSpec((1, R, K), lambda b, i: (b * NB + i, 0, 0)),
        out_shape=jax.ShapeDtypeStruct((B * NB, R, K), jnp.int32),
        scratch_shapes=[pltpu.VMEM((R, NP), jnp.float32)],
    )(pts_blk, ptsT)


def _dense_kernel(x_ref, wrel_ref, b_ref, wroot_ref, z_ref, r_ref):
    x = x_ref[...]
    z_ref[...] = jax.lax.dot_general(
        x, wrel_ref[...], (((1,), (0,)), ((), ())),
        preferred_element_type=jnp.float32)
    r_ref[...] = (jax.lax.dot_general(
        x, wroot_ref[...], (((1,), (0,)), ((), ())),
        preferred_element_type=jnp.float32) + b_ref[0, :][None, :])


RZ = 512


def _run_dense(x, wrel, bvec, wroot):
    di = x.shape[-1]
    do = wrel.shape[-1]
    return pl.pallas_call(
        _dense_kernel,
        grid=(NT // RZ,),
        in_specs=[
            pl.BlockSpec((RZ, di), lambda i: (i, 0)),
            pl.BlockSpec((di, do), lambda i: (0, 0)),
            pl.BlockSpec((1, do), lambda i: (0, 0)),
            pl.BlockSpec((di, do), lambda i: (0, 0)),
        ],
        out_specs=[
            pl.BlockSpec((RZ, do), lambda i: (i, 0)),
            pl.BlockSpec((RZ, do), lambda i: (i, 0)),
        ],
        out_shape=[
            jax.ShapeDtypeStruct((NT, do), jnp.float32),
            jax.ShapeDtypeStruct((NT, do), jnp.float32),
        ],
    )(x, wrel, bvec, wroot)


DO = 128


def _gather_kernel(idx_hbm, z_hbm, r_hbm, out_hbm, idxall, rows0, rows1,
                   rbuf0, rbuf1, obuf0, obuf1, semz0, semz1, semr0, semr1,
                   semo, *, do, relu):
    wid = lax.axis_index("s") * 2 + lax.axis_index("c")
    nd = do // 16
    nstep = PER_W // G
    npair = nstep // 2
    NG = G * K // 128

    pltpu.sync_copy(idx_hbm.at[pl.ds(wid * PER_W * K, PER_W * K)], idxall)

    for obuf in (obuf0, obuf1):
        for g in range(G):
            for d in range(nd, DO // 16):
                obuf[g, d * 16:(d + 1) * 16] = jnp.zeros((16,), jnp.float32)

    def start(s, rows, semz, rbuf, semr):
        for h in range(NG):
            pltpu.async_copy(
                z_hbm.at[idxall.at[pl.ds(s * G * K + h * 128, 128)]],
                rows.at[pl.ds(h * 128, 128), :], semz)
        pltpu.async_copy(r_hbm.at[pl.ds(wid * PER_W + s * G, G)], rbuf, semr)

    def finish(s, rows, semz, rbuf, semr, obuf):
        for h in range(NG):
            pltpu.make_async_copy(
                z_hbm.at[idxall.at[pl.ds(s * G * K + h * 128, 128)]],
                rows.at[pl.ds(h * 128, 128), :], semz).wait()
        pltpu.make_async_copy(
            r_hbm.at[pl.ds(wid * PER_W + s * G, G)], rbuf, semr).wait()
        for g in range(G):
            for d in range(nd):
                acc = rbuf[g, d * 16:(d + 1) * 16]
                for j in range(K):
                    acc = acc + rows[g * K + j, d * 16:(d + 1) * 16]
                if relu:
                    acc = jnp.maximum(acc, 0.0)
                obuf[g, d * 16:(d + 1) * 16] = acc
        pltpu.async_copy(
            obuf, out_hbm.at[pl.ds(wid * PER_W + s * G, G)], semo)

    def drain_out(s, obuf):
        pltpu.make_async_copy(
            obuf, out_hbm.at[pl.ds(wid * PER_W + s * G, G)], semo).wait()

    start(0, rows0, semz0, rbuf0, semr0)

    def pair(it, carry):
        s0 = it * 2
        start(s0 + 1, rows1, semz1, rbuf1, semr1)

        @pl.when(it > 0)
        def _():
            drain_out(s0 - 2, obuf0)

        finish(s0, rows0, semz0, rbuf0, semr0, obuf0)

        @pl.when(it + 1 < npair)
        def _():
            start(s0 + 2, rows0, semz0, rbuf0, semr0)

        @pl.when(it > 0)
        def _():
            drain_out(s0 - 1, obuf1)

        finish(s0 + 1, rows1, semz1, rbuf1, semr1, obuf1)
        return carry

    lax.fori_loop(0, npair, pair, 0)
    drain_out(nstep - 2, obuf0)
    drain_out(nstep - 1, obuf1)


def _run_gather(idx_flat, z, r, *, do, relu):
    mesh = plsc.VectorSubcoreMesh(core_axis_name="c", subcore_axis_name="s")
    body = functools.partial(_gather_kernel, do=do, relu=relu)
    f = functools.partial(
        pl.kernel, mesh=mesh,
        out_type=jax.ShapeDtypeStruct((NT, DO), jnp.float32),
        scratch_types=[
            pltpu.VMEM((PER_W * K,), jnp.int32),
            pltpu.VMEM((G * K, DO), jnp.float32),
            pltpu.VMEM((G * K, DO), jnp.float32),
            pltpu.VMEM((G, DO), jnp.float32),
            pltpu.VMEM((G, DO), jnp.float32),
            pltpu.VMEM((G, DO), jnp.float32),
            pltpu.VMEM((G, DO), jnp.float32),
            pltpu.SemaphoreType.DMA,
            pltpu.SemaphoreType.DMA,
            pltpu.SemaphoreType.DMA,
            pltpu.SemaphoreType.DMA,
            pltpu.SemaphoreType.DMA,
        ],
    )(body)
    return f(idx_flat, z, r)


def kernel(point_cloud, W_rel1, b1, W_root1, W_rel2, b2, W_root2,
           W_rel3, b3, W_root3):
    pts_pad = jnp.pad(point_cloud, ((0, 0), (0, NP - N), (0, DP - 3)))
    ptsT = jnp.transpose(pts_pad, (0, 2, 1))
    pts_blk = pts_pad.reshape(B * NB, R, DP)

    idx = _run_knn(pts_blk, ptsT)
    idx_flat = idx.reshape(NT * K)

    def padw(w, dip):
        return jnp.pad(w, ((0, dip - w.shape[0]), (0, DO - w.shape[1])))

    layers = (
        (padw(W_rel1, DP), jnp.pad(b1, (0, DO - b1.shape[0])),
         padw(W_root1, DP), True),
        (padw(W_rel2, DO), jnp.pad(b2, (0, DO - b2.shape[0])),
         padw(W_root2, DO), True),
        (padw(W_rel3, DO), jnp.pad(b3, (0, DO - b3.shape[0])),
         padw(W_root3, DO), False),
    )
    dos = (64, 64, 128)

    x = pts_pad.reshape(NT, DP)
    for (wrel, bvec, wroot, relu), do in zip(layers, dos):
        z, r = _run_dense(x, wrel, bvec[None, :], wroot)
        x = _run_gather(idx_flat, z, r, do=do, relu=relu)
    return x.reshape(B, NP, DO)[:, :N, :]
